# baseline (speedup 1.0000x reference)
import functools

import jax
import jax.numpy as jnp
from jax import lax
from jax.experimental import pallas as pl
from jax.experimental.pallas import tpu as pltpu
from jax.experimental.pallas import tpu_sc as plsc

B = 16384
NC, NS = 2, 16
NW = NC * NS
B_PER_W = B // NW

BLK = 4096
BF16 = jnp.bfloat16
F32 = jnp.float32


def _sc_gather_both(brand_tab128, bidx, color_tab128, cidx):
    mesh = plsc.VectorSubcoreMesh(core_axis_name="c", subcore_axis_name="s")

    @functools.partial(
        pl.kernel,
        mesh=mesh,
        out_type=(
            jax.ShapeDtypeStruct((B, 128), F32),
            jax.ShapeDtypeStruct((B, 128), F32),
        ),
        scratch_types=[
            pltpu.VMEM((B_PER_W,), jnp.int32),
            pltpu.VMEM((B_PER_W, 128), F32),
            pltpu.SemaphoreType.DMA,
        ],
    )
    def k(btab_hbm, bidx_hbm, ctab_hbm, cidx_hbm, be_hbm, ce_hbm,
          idx_v, rows_v, sem):
        wid = lax.axis_index("s") * NC + lax.axis_index("c")
        base = wid * B_PER_W
        pltpu.sync_copy(bidx_hbm.at[pl.ds(base, B_PER_W)], idx_v)
        pltpu.async_copy(btab_hbm.at[idx_v], rows_v, sem).wait()
        pltpu.sync_copy(rows_v, be_hbm.at[pl.ds(base, B_PER_W)])
        pltpu.sync_copy(cidx_hbm.at[pl.ds(base, B_PER_W)], idx_v)
        pltpu.async_copy(ctab_hbm.at[idx_v], rows_v, sem).wait()
        pltpu.sync_copy(rows_v, ce_hbm.at[pl.ds(base, B_PER_W)])

    return k(brand_tab128, bidx, color_tab128, cidx)


GSTRIDE = 25088
GSTEPS = 49
GROWS = 512


def _pack_brand(r0, r1, r2, r3, out_ref):
    out_ref[...] = jnp.concatenate(
        [r[...] for r in (r0, r1, r2, r3)], axis=1)


def _pack_brand_table(brand_table):
    spec = lambda a: pl.BlockSpec(
        (GROWS, 32), lambda i, a=a: (a * GSTEPS + i, 0))
    return pl.pallas_call(
        _pack_brand,
        grid=(GSTEPS,),
        in_specs=[spec(0), spec(1), spec(2), spec(3)],
        out_specs=pl.BlockSpec((GROWS, 128), lambda i: (i, 0)),
        out_shape=jax.ShapeDtypeStruct((GSTRIDE, 128), F32),
    )(brand_table, brand_table, brand_table, brand_table)


def _tc1(text_ref, price_t_ref, w1all_ref, wp_ref, b1_ref, g_ref):
    xb = text_ref[...].astype(BF16)
    pb = price_t_ref[...].astype(BF16)
    w1t = w1all_ref[0:384, :]
    w1p = w1all_ref[432:448, :]
    pe = lax.dot_general(pb, wp_ref[...], (((0,), (0,)), ((), ())),
                         preferred_element_type=F32)
    g = jnp.dot(xb, w1t, preferred_element_type=F32)
    g += jnp.dot(pe.astype(BF16), w1p, preferred_element_type=F32)
    g_ref[...] = (g + b1_ref[...]).astype(BF16)


def _tc2(g_ref, be128_ref, ce128_ref, bid_ref, cid_ref,
         w1b4_ref, w1c8_ref, w2_ref, b2_ref, out_ref):
    n = be128_ref.shape[0]
    bsel = lax.div(bid_ref[...], GSTRIDE).reshape(n, 1)
    csel = lax.rem(cid_ref[...], 8).reshape(n, 1)
    lane = lax.broadcasted_iota(jnp.int32, (n, 128), 1)
    bemask = jnp.where(lane // 32 == bsel, be128_ref[...], 0.0).astype(BF16)
    cemask = jnp.where(lane // 16 == csel, ce128_ref[...], 0.0).astype(BF16)

    h = g_ref[...].astype(F32)
    h += jnp.dot(bemask, w1b4_ref[...], preferred_element_type=F32)
    h += jnp.dot(cemask, w1c8_ref[...], preferred_element_type=F32)
    h = jnp.maximum(h, 0.0)
    z = jnp.dot(h.astype(BF16), w2_ref[...], preferred_element_type=F32)
    z += b2_ref[...]
    ssq = jnp.maximum(jnp.sum(z * z, axis=1, keepdims=True), 1e-24)
    out_ref[...] = z * lax.rsqrt(ssq)


def kernel(text_emb, brand_id, color_id, price_oneh, brand_table, color_table,
           W_price, W1, b1, W2, b2):
    btab128 = _pack_brand_table(brand_table)
    ctab128 = color_table.reshape(-1, 128)
    bidx = lax.rem(brand_id, GSTRIDE)
    cidx = lax.div(color_id, 8)

    be128, ce128 = _sc_gather_both(btab128, bidx, ctab128, cidx)

    w1all = W1.T.astype(BF16)
    w1b4 = jnp.tile(W1[:, 384:416].T.astype(BF16), (4, 1))
    w1c8 = jnp.tile(W1[:, 416:432].T.astype(BF16), (8, 1))
    wp = W_price.T.astype(BF16)
    w2 = W2.T.astype(BF16)
    b1r = b1.reshape(1, 256)
    b2r = b2.reshape(1, 128)

    grid = (B // BLK,)
    row_spec = lambda w: pl.BlockSpec((BLK, w), lambda i: (i, 0))
    vec_spec = pl.BlockSpec((BLK,), lambda i: (i,))
    full_spec = lambda a, b: pl.BlockSpec((a, b), lambda i: (0, 0))

    g = pl.pallas_call(
        _tc1,
        grid=grid,
        in_specs=[
            row_spec(384),
            pl.BlockSpec((100, BLK), lambda i: (0, i)),
            full_spec(448, 256),
            full_spec(100, 16),
            full_spec(1, 256),
        ],
        out_specs=row_spec(256),
        out_shape=jax.ShapeDtypeStruct((B, 256), BF16),
    )(text_emb, price_oneh.T, w1all, wp, b1r)

    out = pl.pallas_call(
        _tc2,
        grid=grid,
        in_specs=[
            row_spec(256),
            row_spec(128),
            row_spec(128),
            vec_spec,
            vec_spec,
            full_spec(128, 256),
            full_spec(128, 256),
            full_spec(256, 128),
            full_spec(1, 128),
        ],
        out_specs=row_spec(128),
        out_shape=jax.ShapeDtypeStruct((B, 128), F32),
    )(g, be128, ce128, brand_id, color_id, w1b4, w1c8, w2, b2r)
    return out


# speedup vs baseline: 1.2952x; 1.2952x over previous
import functools

import jax
import jax.numpy as jnp
from jax import lax
from jax.experimental import pallas as pl
from jax.experimental.pallas import tpu as pltpu
from jax.experimental.pallas import tpu_sc as plsc

B = 16384
NC, NS = 2, 16
NW = NC * NS
B_PER_W = B // NW

BLK = 4096
BF16 = jnp.bfloat16
F32 = jnp.float32


def _sc_gather_both(brand_tab128, bidx, color_tab128, cidx):
    mesh = plsc.VectorSubcoreMesh(core_axis_name="c", subcore_axis_name="s")

    @functools.partial(
        pl.kernel,
        mesh=mesh,
        out_type=(
            jax.ShapeDtypeStruct((B, 128), F32),
            jax.ShapeDtypeStruct((B, 128), F32),
        ),
        scratch_types=[
            pltpu.VMEM((B_PER_W,), jnp.int32),
            pltpu.VMEM((B_PER_W, 128), F32),
            pltpu.SemaphoreType.DMA,
        ],
    )
    def k(btab_hbm, bidx_hbm, ctab_hbm, cidx_hbm, be_hbm, ce_hbm,
          idx_v, rows_v, sem):
        wid = lax.axis_index("s") * NC + lax.axis_index("c")
        base = wid * B_PER_W
        pltpu.sync_copy(bidx_hbm.at[pl.ds(base, B_PER_W)], idx_v)
        pltpu.async_copy(btab_hbm.at[idx_v], rows_v, sem).wait()
        pltpu.sync_copy(rows_v, be_hbm.at[pl.ds(base, B_PER_W)])
        pltpu.sync_copy(cidx_hbm.at[pl.ds(base, B_PER_W)], idx_v)
        pltpu.async_copy(ctab_hbm.at[idx_v], rows_v, sem).wait()
        pltpu.sync_copy(rows_v, ce_hbm.at[pl.ds(base, B_PER_W)])

    return k(brand_tab128, bidx, color_tab128, cidx)


GSTRIDE = 25088
GSTEPS = 49
GROWS = 512


def _pack_brand(r0, r1, r2, r3, out_ref):
    parts = [jnp.transpose(r[...].astype(BF16)).astype(F32)
             for r in (r0, r1, r2, r3)]
    out_ref[...] = jnp.concatenate(parts, axis=1)


def _pack_brand_table(brand_table_t):
    spec = lambda a: pl.BlockSpec(
        (32, GROWS), lambda i, a=a: (0, a * GSTEPS + i))
    return pl.pallas_call(
        _pack_brand,
        grid=(GSTEPS,),
        in_specs=[spec(0), spec(1), spec(2), spec(3)],
        out_specs=pl.BlockSpec((GROWS, 128), lambda i: (i, 0)),
        out_shape=jax.ShapeDtypeStruct((GSTRIDE, 128), F32),
    )(brand_table_t, brand_table_t, brand_table_t, brand_table_t)


def _tc1(text_ref, price_t_ref, w1all_ref, wp_ref, b1_ref, g_ref):
    xb = text_ref[...].astype(BF16)
    pb = price_t_ref[...].astype(BF16)
    w1t = w1all_ref[0:384, :]
    w1p = w1all_ref[432:448, :]
    pe = lax.dot_general(pb, wp_ref[...], (((0,), (0,)), ((), ())),
                         preferred_element_type=F32)
    g = jnp.dot(xb, w1t, preferred_element_type=F32)
    g += jnp.dot(pe.astype(BF16), w1p, preferred_element_type=F32)
    g_ref[...] = (g + b1_ref[...]).astype(BF16)


def _tc2(g_ref, be128_ref, ce128_ref, bid_ref, cid_ref,
         w1b4_ref, w1c8_ref, w2_ref, b2_ref, out_ref):
    n = be128_ref.shape[0]
    bsel = lax.div(bid_ref[...], GSTRIDE).reshape(n, 1)
    csel = lax.rem(cid_ref[...], 8).reshape(n, 1)
    lane = lax.broadcasted_iota(jnp.int32, (n, 128), 1)
    bemask = jnp.where(lane // 32 == bsel, be128_ref[...], 0.0).astype(BF16)
    cemask = jnp.where(lane // 16 == csel, ce128_ref[...], 0.0).astype(BF16)

    h = g_ref[...].astype(F32)
    h += jnp.dot(bemask, w1b4_ref[...], preferred_element_type=F32)
    h += jnp.dot(cemask, w1c8_ref[...], preferred_element_type=F32)
    h = jnp.maximum(h, 0.0)
    z = jnp.dot(h.astype(BF16), w2_ref[...], preferred_element_type=F32)
    z += b2_ref[...]
    ssq = jnp.maximum(jnp.sum(z * z, axis=1, keepdims=True), 1e-24)
    out_ref[...] = z * lax.rsqrt(ssq)


def kernel(text_emb, brand_id, color_id, price_oneh, brand_table, color_table,
           W_price, W1, b1, W2, b2):
    btab128 = _pack_brand_table(brand_table.T)
    ctab128 = color_table.reshape(-1, 128)
    bidx = lax.rem(brand_id, GSTRIDE)
    cidx = lax.div(color_id, 8)

    be128, ce128 = _sc_gather_both(btab128, bidx, ctab128, cidx)

    w1all = W1.T.astype(BF16)
    w1b4 = jnp.tile(W1[:, 384:416].T.astype(BF16), (4, 1))
    w1c8 = jnp.tile(W1[:, 416:432].T.astype(BF16), (8, 1))
    wp = W_price.T.astype(BF16)
    w2 = W2.T.astype(BF16)
    b1r = b1.reshape(1, 256)
    b2r = b2.reshape(1, 128)

    grid = (B // BLK,)
    row_spec = lambda w: pl.BlockSpec((BLK, w), lambda i: (i, 0))
    vec_spec = pl.BlockSpec((BLK,), lambda i: (i,))
    full_spec = lambda a, b: pl.BlockSpec((a, b), lambda i: (0, 0))

    g = pl.pallas_call(
        _tc1,
        grid=grid,
        in_specs=[
            row_spec(384),
            pl.BlockSpec((100, BLK), lambda i: (0, i)),
            full_spec(448, 256),
            full_spec(100, 16),
            full_spec(1, 256),
        ],
        out_specs=row_spec(256),
        out_shape=jax.ShapeDtypeStruct((B, 256), BF16),
    )(text_emb, price_oneh.T, w1all, wp, b1r)

    out = pl.pallas_call(
        _tc2,
        grid=grid,
        in_specs=[
            row_spec(256),
            row_spec(128),
            row_spec(128),
            vec_spec,
            vec_spec,
            full_spec(128, 256),
            full_spec(128, 256),
            full_spec(256, 128),
            full_spec(1, 128),
        ],
        out_specs=row_spec(128),
        out_shape=jax.ShapeDtypeStruct((B, 128), F32),
    )(g, be128, ce128, brand_id, color_id, w1b4, w1c8, w2, b2r)
    return out
